# baseline (speedup 1.0000x reference)
import functools

import jax
import jax.numpy as jnp
import numpy as np
from jax import lax
from jax.experimental import pallas as pl
from jax.experimental.pallas import tpu as pltpu
from jax.experimental.pallas import tpu_sc as plsc

_NEG_INF = np.float32(-np.inf)
_TINY = np.float32(np.finfo(np.float32).tiny)
_LOG_CLAMP = np.float32(np.log(1e-12))


def _threefry_bits(x1):
    ks0 = np.uint32(0)
    ks1 = np.uint32(42)
    ks2 = np.uint32(ks0 ^ ks1 ^ np.uint32(0x1BD11BDA))

    def rotl(x, d):
        return (x << np.uint32(d)) | (x >> np.uint32(32 - d))

    def rounds(x0, x1, rots):
        for r in rots:
            x0 = x0 + x1
            x1 = rotl(x1, r)
            x1 = x0 ^ x1
        return x0, x1

    x0 = x1
    x1 = rotl(x1, 13) ^ x1
    x0, x1 = rounds(x0, x1, (15, 26, 6))
    x0 = x0 + ks1
    x1 = x1 + (ks2 + np.uint32(1))
    x0, x1 = rounds(x0, x1, (17, 29, 16, 24))
    x0 = x0 + ks2
    x1 = x1 + (ks0 + np.uint32(2))
    x0, x1 = rounds(x0, x1, (13, 15, 26, 6))
    x1 = x1 + (ks1 + np.uint32(3))
    x0, x1 = rounds(x0, x1, (17, 29, 16, 24))
    x0 = x0 + ks1
    x1 = x1 + (ks2 + np.uint32(4))
    x0, x1 = rounds(x0, x1, (13, 15, 26, 6))
    x0 = x0 + ks2
    x1 = x1 + (ks0 + np.uint32(5))
    return x0 ^ x1


def _uniform_from_bits(bits):
    fb = (bits >> np.uint32(9)) | np.uint32(0x3F800000)
    f = jax.lax.bitcast_convert_type(fb, jnp.float32) - np.float32(1.0)
    return jnp.maximum(_TINY, f + _TINY)


_SC_ROWS = 128
_SC_K = 20000


def _sc_u_body(out_hbm, b0, b1, b2, b3):
    n_cores = 2
    wid = lax.axis_index("s") * n_cores + lax.axis_index("c")
    rows_per_w = _SC_ROWS // 32
    lane = lax.iota(jnp.int32, 16)
    n_chunks = 1_000_000 // _SC_K
    bufs = (b0, b1, b2, b3)

    def chunk_body(k, _):
        v0 = k * _SC_K

        def vec_body(i, _):
            off = i * 16
            for r in range(rows_per_w):
                row = wid * rows_per_w + r
                base = (np.int32(384_000_000 + 42) + row * np.int32(1_000_000)
                        + v0 + off)
                x1 = (lane + base).astype(jnp.uint32)
                u = _uniform_from_bits(_threefry_bits(x1))
                bufs[r][pl.ds(off, 16)] = u
            return 0

        lax.fori_loop(0, _SC_K // 16, vec_body, 0)
        for r in range(rows_per_w):
            row = wid * rows_per_w + r
            off = pl.multiple_of(row * np.int32(1_000_000) + v0, 8)
            pltpu.sync_copy(bufs[r], out_hbm.at[pl.ds(off, _SC_K)])
        return 0

    lax.fori_loop(0, n_chunks, chunk_body, 0)


def _make_sc_u():
    mesh = plsc.VectorSubcoreMesh(core_axis_name="c", subcore_axis_name="s")
    return pl.kernel(
        _sc_u_body,
        out_type=jax.ShapeDtypeStruct((_SC_ROWS * 1_000_000,), jnp.float32),
        scratch_types=[pltpu.VMEM((_SC_K,), jnp.float32)] * 4,
        mesh=mesh,
    )


def _tc1_kernel(logits_ref, chosen_ref, scores_ref, logp_ref, logz_ref,
                bz_ref, bi_ref, m_ref, s_ref,
                *, b_rows, v_cols, c_chunk, n_chunks, n_samp):
    j = pl.program_id(0)

    @pl.when(j == 0)
    def _init():
        bz_ref[...] = jnp.full((b_rows, n_samp), _NEG_INF, jnp.float32)
        bi_ref[...] = jnp.zeros((b_rows, n_samp), jnp.int32)
        m_ref[...] = jnp.full((b_rows, 1), _NEG_INF, jnp.float32)
        s_ref[...] = jnp.zeros((b_rows, 1), jnp.float32)

    lb = logits_ref[...]
    col_l = jax.lax.broadcasted_iota(jnp.int32, (b_rows, c_chunk), 1)
    row_base = jax.lax.broadcasted_iota(jnp.int32, (b_rows, c_chunk), 0) * v_cols
    n_base = col_l + row_base

    masked = (v_cols % c_chunk != 0)
    if masked:
        valid = col_l < v_cols - j * c_chunk
        lbm = jnp.where(valid, lb, _NEG_INF)
    else:
        lbm = lb

    m_old = m_ref[...]
    m_new = jnp.maximum(m_old, jnp.max(lbm, axis=1, keepdims=True))
    e = jnp.exp(lbm - m_new)
    s_ref[...] = s_ref[...] * jnp.exp(m_old - m_new) + jnp.sum(
        e, axis=1, keepdims=True)
    m_ref[...] = m_new

    for s in range(n_samp):
        x1 = (n_base + (j * c_chunk + np.int32(s * b_rows * v_cols + 42))
              ).astype(jnp.uint32)
        bits = _threefry_bits(x1)
        u = _uniform_from_bits(bits)
        g = -jnp.log(-jnp.log(u))
        z = g + lbm
        zmax = jnp.max(z, axis=1, keepdims=True)
        eq = z == zmax
        idx_l = jnp.min(jnp.where(eq, col_l, np.int32(0x7FFFFFFF)),
                        axis=1, keepdims=True)
        better = zmax > bz_ref[:, s:s + 1]
        bz_ref[:, s:s + 1] = jnp.where(better, zmax, bz_ref[:, s:s + 1])
        bi_ref[:, s:s + 1] = jnp.where(better, idx_l + j * c_chunk,
                                       bi_ref[:, s:s + 1])

    @pl.when(j == n_chunks - 1)
    def _finish():
        bi = bi_ref[...]
        chosen_ref[...] = bi
        r_iota = jax.lax.broadcasted_iota(jnp.int32, (b_rows, n_samp), 0)
        s_iota = jax.lax.broadcasted_iota(jnp.int32, (b_rows, n_samp), 1)
        n = (s_iota * np.int32(b_rows * v_cols) + r_iota * np.int32(v_cols)
             + bi + np.int32(42)).astype(jnp.uint32)
        g_star = -jnp.log(-jnp.log(_uniform_from_bits(_threefry_bits(n))))
        scores = bz_ref[...] - g_star
        scores_ref[...] = scores
        log_z = m_ref[...] + jnp.log(s_ref[...])
        logz_ref[...] = log_z
        logp_ref[...] = jnp.maximum(scores - log_z, _LOG_CLAMP)


def _tc2_kernel(logits_ref, u_ref, logz_ref, chosen_ref, scores_ref, logp_ref,
                bz_ref, bi_ref,
                *, b_rows, v_cols, c_chunk, n_chunks, s_lo, n_samp):
    j = pl.program_id(0)

    @pl.when(j == 0)
    def _init():
        bz_ref[...] = jnp.full((b_rows, n_samp), _NEG_INF, jnp.float32)
        bi_ref[...] = jnp.zeros((b_rows, n_samp), jnp.int32)

    lb = logits_ref[...]
    col_l = jax.lax.broadcasted_iota(jnp.int32, (b_rows, c_chunk), 1)
    masked = (v_cols % c_chunk != 0)
    if masked:
        valid = col_l < v_cols - j * c_chunk
        lbm = jnp.where(valid, lb, _NEG_INF)
    else:
        lbm = lb

    for s in range(n_samp):
        u = u_ref[s * b_rows:(s + 1) * b_rows, :]
        g = -jnp.log(-jnp.log(u))
        z = g + lbm
        zmax = jnp.max(z, axis=1, keepdims=True)
        eq = z == zmax
        idx_l = jnp.min(jnp.where(eq, col_l, np.int32(0x7FFFFFFF)),
                        axis=1, keepdims=True)
        better = zmax > bz_ref[:, s:s + 1]
        bz_ref[:, s:s + 1] = jnp.where(better, zmax, bz_ref[:, s:s + 1])
        bi_ref[:, s:s + 1] = jnp.where(better, idx_l + j * c_chunk,
                                       bi_ref[:, s:s + 1])

    @pl.when(j == n_chunks - 1)
    def _finish():
        bi = bi_ref[...]
        chosen_ref[...] = bi
        r_iota = jax.lax.broadcasted_iota(jnp.int32, (b_rows, n_samp), 0)
        s_iota = jax.lax.broadcasted_iota(jnp.int32, (b_rows, n_samp), 1)
        n = ((s_iota + np.int32(s_lo)) * np.int32(b_rows * v_cols)
             + r_iota * np.int32(v_cols) + bi + np.int32(42)).astype(jnp.uint32)
        g_star = -jnp.log(-jnp.log(_uniform_from_bits(_threefry_bits(n))))
        scores = bz_ref[...] - g_star
        scores_ref[...] = scores
        logp_ref[...] = jnp.maximum(scores - logz_ref[...], _LOG_CLAMP)


@jax.jit
def kernel(logits):
    b_rows, v_cols = logits.shape
    c_chunk = 1024
    n_chunks = -(-v_cols // c_chunk)
    n1 = 6

    u = _make_sc_u()().reshape(_SC_ROWS, 1_000_000)

    body1 = functools.partial(_tc1_kernel, b_rows=b_rows, v_cols=v_cols,
                              c_chunk=c_chunk, n_chunks=n_chunks, n_samp=n1)
    c1, s1, p1, logz = pl.pallas_call(
        body1,
        grid=(n_chunks,),
        in_specs=[pl.BlockSpec((b_rows, c_chunk), lambda j: (0, j))],
        out_specs=[
            pl.BlockSpec((b_rows, n1), lambda j: (0, 0)),
            pl.BlockSpec((b_rows, n1), lambda j: (0, 0)),
            pl.BlockSpec((b_rows, n1), lambda j: (0, 0)),
            pl.BlockSpec((b_rows, 1), lambda j: (0, 0)),
        ],
        out_shape=[
            jax.ShapeDtypeStruct((b_rows, n1), jnp.int32),
            jax.ShapeDtypeStruct((b_rows, n1), jnp.float32),
            jax.ShapeDtypeStruct((b_rows, n1), jnp.float32),
            jax.ShapeDtypeStruct((b_rows, 1), jnp.float32),
        ],
        scratch_shapes=[
            pltpu.VMEM((b_rows, n1), jnp.float32),
            pltpu.VMEM((b_rows, n1), jnp.int32),
            pltpu.VMEM((b_rows, 1), jnp.float32),
            pltpu.VMEM((b_rows, 1), jnp.float32),
        ],
        compiler_params=pltpu.CompilerParams(
            dimension_semantics=("arbitrary",),
        ),
    )(logits)

    n2 = 2
    body2 = functools.partial(_tc2_kernel, b_rows=b_rows, v_cols=v_cols,
                              c_chunk=c_chunk, n_chunks=n_chunks,
                              s_lo=n1, n_samp=n2)
    c2, s2, p2 = pl.pallas_call(
        body2,
        grid=(n_chunks,),
        in_specs=[
            pl.BlockSpec((b_rows, c_chunk), lambda j: (0, j)),
            pl.BlockSpec((n2 * b_rows, c_chunk), lambda j: (0, j)),
            pl.BlockSpec((b_rows, 1), lambda j: (0, 0)),
        ],
        out_specs=[
            pl.BlockSpec((b_rows, n2), lambda j: (0, 0)),
            pl.BlockSpec((b_rows, n2), lambda j: (0, 0)),
            pl.BlockSpec((b_rows, n2), lambda j: (0, 0)),
        ],
        out_shape=[
            jax.ShapeDtypeStruct((b_rows, n2), jnp.int32),
            jax.ShapeDtypeStruct((b_rows, n2), jnp.float32),
            jax.ShapeDtypeStruct((b_rows, n2), jnp.float32),
        ],
        scratch_shapes=[
            pltpu.VMEM((b_rows, n2), jnp.float32),
            pltpu.VMEM((b_rows, n2), jnp.int32),
        ],
        compiler_params=pltpu.CompilerParams(
            dimension_semantics=("arbitrary",),
        ),
    )(logits, u, logz)

    chosen = jnp.concatenate([c1, c2], axis=1)
    scores = jnp.concatenate([s1, s2], axis=1)
    logp = jnp.concatenate([p1, p2], axis=1)
    return (chosen, scores, logp)


# speedup vs baseline: 2.3804x; 2.3804x over previous
import functools

import jax
import jax.numpy as jnp
import numpy as np
from jax.experimental import pallas as pl
from jax.experimental.pallas import tpu as pltpu

_NS = 8
_NEG_INF = np.float32(-np.inf)
_TINY = np.float32(np.finfo(np.float32).tiny)
_LOG_CLAMP = np.float32(np.log(1e-12))


def _threefry_bits(x1):
    ks0 = np.uint32(0)
    ks1 = np.uint32(42)
    ks2 = np.uint32(ks0 ^ ks1 ^ np.uint32(0x1BD11BDA))

    def rotl(x, d):
        return (x << np.uint32(d)) | (x >> np.uint32(32 - d))

    def rounds(x0, x1, rots):
        for r in rots:
            x0 = x0 + x1
            x1 = rotl(x1, r)
            x1 = x0 ^ x1
        return x0, x1

    x0 = x1
    x1 = rotl(x1, 13) ^ x1
    x0, x1 = rounds(x0, x1, (15, 26, 6))
    x0 = x0 + ks1
    x1 = x1 + (ks2 + np.uint32(1))
    x0, x1 = rounds(x0, x1, (17, 29, 16, 24))
    x0 = x0 + ks2
    x1 = x1 + (ks0 + np.uint32(2))
    x0, x1 = rounds(x0, x1, (13, 15, 26, 6))
    x1 = x1 + (ks1 + np.uint32(3))
    x0, x1 = rounds(x0, x1, (17, 29, 16, 24))
    x0 = x0 + ks1
    x1 = x1 + (ks2 + np.uint32(4))
    x0, x1 = rounds(x0, x1, (13, 15, 26, 6))
    x0 = x0 + ks2
    x1 = x1 + (ks0 + np.uint32(5))
    return x0 ^ x1


def _gumbel_from_bits(bits):
    fb = (bits >> np.uint32(9)) | np.uint32(0x3F800000)
    f = jax.lax.bitcast_convert_type(fb, jnp.float32) - np.float32(1.0)
    u = f + _TINY
    return -jnp.log(-jnp.log(u))


def _sample_kernel(logits_ref, chosen_ref, scores_ref, logp_ref,
                   bz_ref, bi_ref, m_ref, s_ref,
                   *, b_rows, v_cols, c_chunk, n_chunks):
    j = pl.program_id(0)

    @pl.when(j == 0)
    def _init():
        bz_ref[...] = jnp.full((b_rows, _NS), _NEG_INF, jnp.float32)
        bi_ref[...] = jnp.zeros((b_rows, _NS), jnp.int32)
        m_ref[...] = jnp.full((b_rows, 1), _NEG_INF, jnp.float32)
        s_ref[...] = jnp.zeros((b_rows, 1), jnp.float32)

    lb = logits_ref[...]
    col_l = jax.lax.broadcasted_iota(jnp.int32, (b_rows, c_chunk), 1)
    row_base = jax.lax.broadcasted_iota(jnp.int32, (b_rows, c_chunk), 0) * v_cols
    n_base = col_l + row_base

    def scan_block(masked):
        if masked:
            valid = col_l < v_cols - j * c_chunk
            lbm = jnp.where(valid, lb, _NEG_INF)
        else:
            lbm = lb

        m_old = m_ref[...]
        m_new = jnp.maximum(m_old, jnp.max(lbm, axis=1, keepdims=True))
        e = jnp.exp(lbm - m_new)
        s_ref[...] = s_ref[...] * jnp.exp(m_old - m_new) + jnp.sum(
            e, axis=1, keepdims=True)
        m_ref[...] = m_new

        for s in range(_NS):
            x1 = (n_base + (j * c_chunk + np.int32(s * b_rows * v_cols + 42))
                  ).astype(jnp.uint32)
            g = _gumbel_from_bits(_threefry_bits(x1))
            z = g + lbm
            zmax = jnp.max(z, axis=1, keepdims=True)
            eq = z == zmax
            idx_l = jnp.min(jnp.where(eq, col_l, np.int32(0x7FFFFFFF)),
                            axis=1, keepdims=True)
            better = zmax > bz_ref[:, s:s + 1]
            bz_ref[:, s:s + 1] = jnp.where(better, zmax, bz_ref[:, s:s + 1])
            bi_ref[:, s:s + 1] = jnp.where(better, idx_l + j * c_chunk,
                                           bi_ref[:, s:s + 1])

    scan_block(masked=(v_cols % c_chunk != 0))

    @pl.when(j == n_chunks - 1)
    def _finish():
        bi = bi_ref[...]
        chosen_ref[...] = bi
        r_iota = jax.lax.broadcasted_iota(jnp.int32, (b_rows, _NS), 0)
        s_iota = jax.lax.broadcasted_iota(jnp.int32, (b_rows, _NS), 1)
        n = (s_iota * np.int32(b_rows * v_cols) + r_iota * np.int32(v_cols)
             + bi + np.int32(42)).astype(jnp.uint32)
        g_star = _gumbel_from_bits(_threefry_bits(n))
        scores = bz_ref[...] - g_star
        scores_ref[...] = scores
        log_z = m_ref[...] + jnp.log(s_ref[...])
        logp_ref[...] = jnp.maximum(scores - log_z, _LOG_CLAMP)


@jax.jit
def kernel(logits):
    b_rows, v_cols = logits.shape
    c_chunk = 1024
    n_chunks = -(-v_cols // c_chunk)

    body = functools.partial(_sample_kernel, b_rows=b_rows, v_cols=v_cols,
                             c_chunk=c_chunk, n_chunks=n_chunks)
    chosen, scores, logp = pl.pallas_call(
        body,
        grid=(n_chunks,),
        in_specs=[pl.BlockSpec((b_rows, c_chunk), lambda j: (0, j))],
        out_specs=[
            pl.BlockSpec((b_rows, _NS), lambda j: (0, 0)),
            pl.BlockSpec((b_rows, _NS), lambda j: (0, 0)),
            pl.BlockSpec((b_rows, _NS), lambda j: (0, 0)),
        ],
        out_shape=[
            jax.ShapeDtypeStruct((b_rows, _NS), jnp.int32),
            jax.ShapeDtypeStruct((b_rows, _NS), jnp.float32),
            jax.ShapeDtypeStruct((b_rows, _NS), jnp.float32),
        ],
        scratch_shapes=[
            pltpu.VMEM((b_rows, _NS), jnp.float32),
            pltpu.VMEM((b_rows, _NS), jnp.int32),
            pltpu.VMEM((b_rows, 1), jnp.float32),
            pltpu.VMEM((b_rows, 1), jnp.float32),
        ],
        compiler_params=pltpu.CompilerParams(
            dimension_semantics=("arbitrary",),
        ),
    )(logits)
    return (chosen, scores, logp)
